# initial kernel scaffold (unmeasured)
import jax
import jax.numpy as jnp
from jax import lax
from jax.experimental import pallas as pl
from jax.experimental.pallas import tpu as pltpu

N_Z = 4
S = 1024
D = 2048
DCS = 128
H = 16
DH = 128
DR = 32
PAY = S + 2 * D


def kernel(x, Wdkv, Wuk, Wuv, Wq, Wqr, Wkr, Wo):
    def body(x_ref, wdkv_ref, wuk_ref, wuv_ref, wq_ref, wqr_ref, wkr_ref,
             wo_ref, out_ref, comm_ref, k_ref, v_ref, q_ref,
             send_sems, recv_sems):
        my_x = lax.axis_index("x")
        my_y = lax.axis_index("y")
        my_z = lax.axis_index("z")
        left = (my_z - 1) % N_Z
        right = (my_z + 1) % N_Z

        barrier_sem = pltpu.get_barrier_semaphore()
        for nbr in [left, right]:
            pl.semaphore_signal(
                barrier_sem, inc=1,
                device_id=(my_x, my_y, nbr),
                device_id_type=pl.DeviceIdType.MESH,
            )
        pl.semaphore_wait(barrier_sem, 2)

        xx = x_ref[0]

        cT = lax.dot_general(
            wdkv_ref[...], xx, (((0,), (1,)), ((), ())),
            preferred_element_type=jnp.float32,
        )
        comm_ref[0, :, 0:S] = cT
        comm_ref[0, :, S:S + D] = wuk_ref[...]
        comm_ref[0, :, S + D:PAY] = wuv_ref[...]

        k_ref[...] = lax.dot_general(
            cT, wuk_ref[...], (((0,), (0,)), ((), ())),
            preferred_element_type=jnp.float32,
        )
        v_ref[...] = lax.dot_general(
            cT, wuv_ref[...], (((0,), (0,)), ((), ())),
            preferred_element_type=jnp.float32,
        )

        for h in range(N_Z - 1):
            send_slot = h % 2
            recv_slot = (h + 1) % 2
            rdma = pltpu.make_async_remote_copy(
                src_ref=comm_ref.at[send_slot],
                dst_ref=comm_ref.at[recv_slot],
                send_sem=send_sems.at[send_slot],
                recv_sem=recv_sems.at[recv_slot],
                device_id=(my_x, my_y, right),
                device_id_type=pl.DeviceIdType.MESH,
            )
            rdma.start()
            rdma.wait()

            chunk_cT = comm_ref[recv_slot, :, 0:S]
            k_ref[...] += lax.dot_general(
                chunk_cT, comm_ref[recv_slot, :, S:S + D],
                (((0,), (0,)), ((), ())),
                preferred_element_type=jnp.float32,
            )
            v_ref[...] += lax.dot_general(
                chunk_cT, comm_ref[recv_slot, :, S + D:PAY],
                (((0,), (0,)), ((), ())),
                preferred_element_type=jnp.float32,
            )

        q_ref[...] = jnp.dot(xx, wq_ref[...], preferred_element_type=jnp.float32)
        qr = jnp.dot(xx, wqr_ref[...], preferred_element_type=jnp.float32)
        kr = jnp.dot(xx, wkr_ref[...], preferred_element_type=jnp.float32)

        scale = (DH + DR) ** -0.5
        for h in range(H):
            q_h = q_ref[:, h * DH:(h + 1) * DH]
            k_h = k_ref[:, h * DH:(h + 1) * DH]
            s = lax.dot_general(
                q_h, k_h, (((1,), (1,)), ((), ())),
                preferred_element_type=jnp.float32,
            )
            s += lax.dot_general(
                qr[:, h * DR:(h + 1) * DR], kr, (((1,), (1,)), ((), ())),
                preferred_element_type=jnp.float32,
            )
            s *= scale
            m = jnp.max(s, axis=1, keepdims=True)
            p = jnp.exp(s - m)
            p = p / jnp.sum(p, axis=1, keepdims=True)
            q_ref[:, h * DH:(h + 1) * DH] = jnp.dot(
                p, v_ref[:, h * DH:(h + 1) * DH],
                preferred_element_type=jnp.float32,
            )

        out_ref[0] = jnp.dot(
            q_ref[...], wo_ref[...], preferred_element_type=jnp.float32
        )

    return pl.pallas_call(
        body,
        out_shape=jax.ShapeDtypeStruct((1, S, D), jnp.float32),
        in_specs=[pl.BlockSpec(memory_space=pltpu.VMEM)] * 8,
        out_specs=pl.BlockSpec(memory_space=pltpu.VMEM),
        scratch_shapes=[
            pltpu.VMEM((2, DCS, PAY), jnp.float32),
            pltpu.VMEM((S, D), jnp.float32),
            pltpu.VMEM((S, D), jnp.float32),
            pltpu.VMEM((S, D), jnp.float32),
            pltpu.SemaphoreType.DMA((2,)),
            pltpu.SemaphoreType.DMA((2,)),
        ],
        compiler_params=pltpu.CompilerParams(collective_id=0),
    )(x, Wdkv, Wuk, Wuv, Wq, Wqr, Wkr, Wo)


# baseline (device time: 222737 ns/iter reference)
import jax
import jax.numpy as jnp
from jax import lax
from jax.experimental import pallas as pl
from jax.experimental.pallas import tpu as pltpu

N_Z = 4
S = 1024
D = 2048
DCS = 128
H = 16
DH = 128
DR = 32
PAY = S + 2 * D
SCALE = (DH + DR) ** -0.5

_F32 = jnp.float32


def _kv_body(x_ref, wdkv_ref, wuk_ref, wuv_ref, k_ref, v_ref,
             comm_ref, send_sems, recv_sems):
    my_x = lax.axis_index("x")
    my_y = lax.axis_index("y")
    my_z = lax.axis_index("z")
    left = (my_z - 1) % N_Z
    right = (my_z + 1) % N_Z

    barrier_sem = pltpu.get_barrier_semaphore()
    for nbr in [left, right]:
        pl.semaphore_signal(
            barrier_sem, inc=1,
            device_id=(my_x, my_y, nbr),
            device_id_type=pl.DeviceIdType.MESH,
        )
    pl.semaphore_wait(barrier_sem, 2)

    xx = x_ref[0]

    cT = lax.dot_general(
        wdkv_ref[...], xx, (((0,), (1,)), ((), ())),
        preferred_element_type=_F32,
    )
    comm_ref[0, :, 0:S] = cT
    comm_ref[0, :, S:S + D] = wuk_ref[...]
    comm_ref[0, :, S + D:PAY] = wuv_ref[...]

    k_ref[...] = lax.dot_general(
        cT, wuk_ref[...], (((0,), (0,)), ((), ())),
        preferred_element_type=_F32,
    )
    v_ref[...] = lax.dot_general(
        cT, wuv_ref[...], (((0,), (0,)), ((), ())),
        preferred_element_type=_F32,
    )

    for h in range(N_Z - 1):
        send_slot = h % 2
        recv_slot = (h + 1) % 2
        rdma = pltpu.make_async_remote_copy(
            src_ref=comm_ref.at[send_slot],
            dst_ref=comm_ref.at[recv_slot],
            send_sem=send_sems.at[send_slot],
            recv_sem=recv_sems.at[recv_slot],
            device_id=(my_x, my_y, right),
            device_id_type=pl.DeviceIdType.MESH,
        )
        rdma.start()
        rdma.wait()

        chunk_cT = comm_ref[recv_slot, :, 0:S]
        k_ref[...] += lax.dot_general(
            chunk_cT, comm_ref[recv_slot, :, S:S + D],
            (((0,), (0,)), ((), ())),
            preferred_element_type=_F32,
        )
        v_ref[...] += lax.dot_general(
            chunk_cT, comm_ref[recv_slot, :, S + D:PAY],
            (((0,), (0,)), ((), ())),
            preferred_element_type=_F32,
        )


def _proj_body(x_ref, wq_ref, wqr_ref, wkr_ref, q_ref, qr_ref, kr_ref):
    xx = x_ref[0]
    q_ref[...] = jnp.dot(xx, wq_ref[...], preferred_element_type=_F32)
    qr = jnp.dot(xx, wqr_ref[...], preferred_element_type=_F32)
    for h in range(H):
        qr_ref[h] = qr[:, h * DR:(h + 1) * DR]
    kr_ref[...] = jnp.dot(xx, wkr_ref[...], preferred_element_type=_F32)


def _attn_body(q_ref, k_ref, v_ref, qr_ref, kr_ref, o_ref):
    s = lax.dot_general(
        q_ref[...], k_ref[...], (((1,), (1,)), ((), ())),
        preferred_element_type=_F32,
    )
    s += lax.dot_general(
        qr_ref[0], kr_ref[...], (((1,), (1,)), ((), ())),
        preferred_element_type=_F32,
    )
    s *= SCALE
    m = jnp.max(s, axis=1, keepdims=True)
    p = jnp.exp(s - m)
    p = p / jnp.sum(p, axis=1, keepdims=True)
    o_ref[...] = jnp.dot(p, v_ref[...], preferred_element_type=_F32)


def _out_body(o_ref, wo_ref, out_ref):
    out_ref[0] = jnp.dot(o_ref[...], wo_ref[...], preferred_element_type=_F32)


def kernel(x, Wdkv, Wuk, Wuv, Wq, Wqr, Wkr, Wo):
    vmem = pl.BlockSpec(memory_space=pltpu.VMEM)

    K, V = pl.pallas_call(
        _kv_body,
        out_shape=(
            jax.ShapeDtypeStruct((S, D), _F32),
            jax.ShapeDtypeStruct((S, D), _F32),
        ),
        in_specs=[vmem] * 4,
        out_specs=(vmem, vmem),
        scratch_shapes=[
            pltpu.VMEM((2, DCS, PAY), _F32),
            pltpu.SemaphoreType.DMA((2,)),
            pltpu.SemaphoreType.DMA((2,)),
        ],
        compiler_params=pltpu.CompilerParams(collective_id=0),
    )(x, Wdkv, Wuk, Wuv)

    Q, Qr, Kr = pl.pallas_call(
        _proj_body,
        out_shape=(
            jax.ShapeDtypeStruct((S, D), _F32),
            jax.ShapeDtypeStruct((H, S, DR), _F32),
            jax.ShapeDtypeStruct((S, DR), _F32),
        ),
        in_specs=[vmem] * 4,
        out_specs=(vmem, vmem, vmem),
    )(x, Wq, Wqr, Wkr)

    O = pl.pallas_call(
        _attn_body,
        grid=(H,),
        out_shape=jax.ShapeDtypeStruct((S, D), _F32),
        in_specs=[
            pl.BlockSpec((S, DH), lambda h: (0, h)),
            pl.BlockSpec((S, DH), lambda h: (0, h)),
            pl.BlockSpec((S, DH), lambda h: (0, h)),
            pl.BlockSpec((1, S, DR), lambda h: (h, 0, 0)),
            pl.BlockSpec((S, DR), lambda h: (0, 0)),
        ],
        out_specs=pl.BlockSpec((S, DH), lambda h: (0, h)),
    )(Q, K, V, Qr, Kr)

    return pl.pallas_call(
        _out_body,
        out_shape=jax.ShapeDtypeStruct((1, S, D), _F32),
        in_specs=[vmem, vmem],
        out_specs=vmem,
    )(O, Wo)


# device time: 220908 ns/iter; 1.0083x vs baseline; 1.0083x over previous
import jax
import jax.numpy as jnp
from jax import lax
from jax.experimental import pallas as pl
from jax.experimental.pallas import tpu as pltpu

N_Z = 4
S = 1024
D = 2048
DCS = 128
H = 16
DH = 128
DR = 32
PAY = S + 2 * D
SCALE = (DH + DR) ** -0.5

_F32 = jnp.float32


def _kv_body(x_ref, wdkv_ref, wuk_ref, wuv_ref,
             k_ref, v_ref, ct_buf, recv_buf, send_sems, recv_sems):
    my_x = lax.axis_index("x")
    my_y = lax.axis_index("y")
    my_z = lax.axis_index("z")
    peers = [(my_z + 1) % N_Z, (my_z + 3) % N_Z, (my_z + 2) % N_Z]

    barrier_sem = pltpu.get_barrier_semaphore()
    for w in peers:
        pl.semaphore_signal(
            barrier_sem, inc=1,
            device_id=(my_x, my_y, w),
            device_id_type=pl.DeviceIdType.MESH,
        )
    pl.semaphore_wait(barrier_sem, N_Z - 1)

    xx = x_ref[0]

    ct_buf[...] = lax.dot_general(
        wdkv_ref[...], xx, (((0,), (1,)), ((), ())),
        preferred_element_type=_F32,
    )

    def _payload(buf, slot):
        return (
            buf.at[slot, :, 0:S],
            buf.at[slot, :, S:S + D],
            buf.at[slot, :, S + D:PAY],
        )

    rdmas = []
    for w in peers:
        dsts = _payload(recv_buf, my_z)
        for j, src in enumerate((ct_buf, wuk_ref, wuv_ref)):
            rdma = pltpu.make_async_remote_copy(
                src_ref=src,
                dst_ref=dsts[j],
                send_sem=send_sems.at[w, j],
                recv_sem=recv_sems.at[my_z, j],
                device_id=(my_x, my_y, w),
                device_id_type=pl.DeviceIdType.MESH,
            )
            rdma.start()
            rdmas.append(rdma)

    cT = ct_buf[...]
    k_ref[...] = lax.dot_general(
        cT, wuk_ref[...], (((0,), (0,)), ((), ())),
        preferred_element_type=_F32,
    )
    v_ref[...] = lax.dot_general(
        cT, wuv_ref[...], (((0,), (0,)), ((), ())),
        preferred_element_type=_F32,
    )

    for s in peers:
        dsts = _payload(recv_buf, s)
        for j, src in enumerate((ct_buf, wuk_ref, wuv_ref)):
            recv = pltpu.make_async_remote_copy(
                src_ref=src,
                dst_ref=dsts[j],
                send_sem=send_sems.at[s, j],
                recv_sem=recv_sems.at[s, j],
                device_id=(my_x, my_y, s),
                device_id_type=pl.DeviceIdType.MESH,
            )
            recv.wait_recv()
        chunk_cT = recv_buf[s, :, 0:S]
        k_ref[...] += lax.dot_general(
            chunk_cT, recv_buf[s, :, S:S + D],
            (((0,), (0,)), ((), ())),
            preferred_element_type=_F32,
        )
        v_ref[...] += lax.dot_general(
            chunk_cT, recv_buf[s, :, S + D:PAY],
            (((0,), (0,)), ((), ())),
            preferred_element_type=_F32,
        )

    for r in rdmas:
        r.wait_send()


def _proj_body(x_ref, wq_ref, wqr_ref, wkr_ref, q_ref, qr_ref, kr_ref):
    xx = x_ref[0]
    q_ref[...] = jnp.dot(xx, wq_ref[...], preferred_element_type=_F32) * SCALE
    qr = jnp.dot(xx, wqr_ref[...], preferred_element_type=_F32) * SCALE
    for h in range(H):
        qr_ref[h] = qr[:, h * DR:(h + 1) * DR]
    kr_ref[...] = jnp.dot(xx, wkr_ref[...], preferred_element_type=_F32)


def _attn_body(q_ref, k_ref, v_ref, qr_ref, kr_ref, o_ref):
    s = lax.dot_general(
        q_ref[...], k_ref[...], (((1,), (1,)), ((), ())),
        preferred_element_type=_F32,
    )
    s += lax.dot_general(
        qr_ref[0], kr_ref[...], (((1,), (1,)), ((), ())),
        preferred_element_type=_F32,
    )
    p = jnp.exp(s)
    denom = jnp.sum(p, axis=1, keepdims=True)
    o = jnp.dot(p, v_ref[...], preferred_element_type=_F32)
    o_ref[...] = o / denom


def _out_body(o_ref, wo_ref, out_ref):
    out_ref[0] = jnp.dot(o_ref[...], wo_ref[...], preferred_element_type=_F32)


def kernel(x, Wdkv, Wuk, Wuv, Wq, Wqr, Wkr, Wo):
    vmem = pl.BlockSpec(memory_space=pltpu.VMEM)

    K, V = pl.pallas_call(
        _kv_body,
        out_shape=(
            jax.ShapeDtypeStruct((S, D), _F32),
            jax.ShapeDtypeStruct((S, D), _F32),
        ),
        in_specs=[vmem] * 4,
        out_specs=(vmem, vmem),
        scratch_shapes=[
            pltpu.VMEM((DCS, S), _F32),
            pltpu.VMEM((N_Z, DCS, PAY), _F32),
            pltpu.SemaphoreType.DMA((N_Z, 3)),
            pltpu.SemaphoreType.DMA((N_Z, 3)),
        ],
        compiler_params=pltpu.CompilerParams(
            collective_id=0, vmem_limit_bytes=34 * 1024 * 1024
        ),
    )(x, Wdkv, Wuk, Wuv)

    Q, Qr, Kr = pl.pallas_call(
        _proj_body,
        out_shape=(
            jax.ShapeDtypeStruct((S, D), _F32),
            jax.ShapeDtypeStruct((H, S, DR), _F32),
            jax.ShapeDtypeStruct((S, DR), _F32),
        ),
        in_specs=[vmem] * 4,
        out_specs=(vmem, vmem, vmem),
    )(x, Wq, Wqr, Wkr)

    O = pl.pallas_call(
        _attn_body,
        grid=(H,),
        out_shape=jax.ShapeDtypeStruct((S, D), _F32),
        in_specs=[
            pl.BlockSpec((S, DH), lambda h: (0, h)),
            pl.BlockSpec((S, DH), lambda h: (0, h)),
            pl.BlockSpec((S, DH), lambda h: (0, h)),
            pl.BlockSpec((1, S, DR), lambda h: (h, 0, 0)),
            pl.BlockSpec((S, DR), lambda h: (0, 0)),
        ],
        out_specs=pl.BlockSpec((S, DH), lambda h: (0, h)),
    )(Q, K, V, Qr, Kr)

    return pl.pallas_call(
        _out_body,
        out_shape=jax.ShapeDtypeStruct((1, S, D), _F32),
        in_specs=[vmem, vmem],
        out_specs=vmem,
    )(O, Wo)


# device time: 153730 ns/iter; 1.4489x vs baseline; 1.4370x over previous
import jax
import jax.numpy as jnp
from jax import lax
from jax.experimental import pallas as pl
from jax.experimental.pallas import tpu as pltpu

N_Z = 4
S = 1024
D = 2048
DCS = 128
H = 16
DH = 128
DR = 32
PAY = S + 2 * D
SCALE = (DH + DR) ** -0.5

_F32 = jnp.float32


def _kv_body(x_ref, wdkv_ref, wuk_ref, wuv_ref,
             k_ref, v_ref, comm_ref, send_sems, recv_sems):
    my_x = lax.axis_index("x")
    my_y = lax.axis_index("y")
    my_z = lax.axis_index("z")
    left = (my_z - 1) % N_Z
    right = (my_z + 1) % N_Z

    barrier_sem = pltpu.get_barrier_semaphore()
    for nbr in [left, right]:
        pl.semaphore_signal(
            barrier_sem, inc=1,
            device_id=(my_x, my_y, nbr),
            device_id_type=pl.DeviceIdType.MESH,
        )
    pl.semaphore_wait(barrier_sem, 2)

    xx = x_ref[0]

    cT = lax.dot_general(
        wdkv_ref[...], xx, (((0,), (1,)), ((), ())),
        preferred_element_type=_F32,
    )
    comm_ref[0, :, 0:S] = cT.astype(jnp.bfloat16)
    comm_ref[0, :, S:S + D] = wuk_ref[...].astype(jnp.bfloat16)
    comm_ref[0, :, S + D:PAY] = wuv_ref[...].astype(jnp.bfloat16)

    def acc(slot):
        chunk_cT = comm_ref[slot, :, 0:S]
        k_ref[...] += lax.dot_general(
            chunk_cT, comm_ref[slot, :, S:S + D],
            (((0,), (0,)), ((), ())),
            preferred_element_type=_F32,
        )
        v_ref[...] += lax.dot_general(
            chunk_cT, comm_ref[slot, :, S + D:PAY],
            (((0,), (0,)), ((), ())),
            preferred_element_type=_F32,
        )

    for h in range(N_Z - 1):
        rdma = pltpu.make_async_remote_copy(
            src_ref=comm_ref.at[h],
            dst_ref=comm_ref.at[(h + 1) % 3],
            send_sem=send_sems.at[h],
            recv_sem=recv_sems.at[(h + 1) % 3],
            device_id=(my_x, my_y, right),
            device_id_type=pl.DeviceIdType.MESH,
        )
        rdma.start()
        if h == 0:
            k_ref[...] = lax.dot_general(
                cT, wuk_ref[...], (((0,), (0,)), ((), ())),
                preferred_element_type=_F32,
            )
            v_ref[...] = lax.dot_general(
                cT, wuv_ref[...], (((0,), (0,)), ((), ())),
                preferred_element_type=_F32,
            )
        else:
            acc(h)
        rdma.wait()
    acc(0)


def _proj_body(x_ref, wq_ref, wqr_ref, wkr_ref, q_ref, qr_ref, kr_ref):
    xx = x_ref[0]
    q_ref[...] = jnp.dot(xx, wq_ref[...], preferred_element_type=_F32) * SCALE
    qr = jnp.dot(xx, wqr_ref[...], preferred_element_type=_F32) * SCALE
    for h in range(H):
        qr_ref[h] = qr[:, h * DR:(h + 1) * DR]
    kr_ref[...] = jnp.dot(xx, wkr_ref[...], preferred_element_type=_F32)


def _attn_body(q_ref, k_ref, v_ref, qr_ref, kr_ref, o_ref):
    s = lax.dot_general(
        q_ref[...], k_ref[...], (((1,), (1,)), ((), ())),
        preferred_element_type=_F32,
    )
    s += lax.dot_general(
        qr_ref[0], kr_ref[...], (((1,), (1,)), ((), ())),
        preferred_element_type=_F32,
    )
    p = jnp.exp(s)
    denom = jnp.sum(p, axis=1, keepdims=True)
    o = jnp.dot(p, v_ref[...], preferred_element_type=_F32)
    o_ref[...] = o / denom


def _out_body(o_ref, wo_ref, out_ref):
    out_ref[0] = jnp.dot(o_ref[...], wo_ref[...], preferred_element_type=_F32)


def kernel(x, Wdkv, Wuk, Wuv, Wq, Wqr, Wkr, Wo):
    vmem = pl.BlockSpec(memory_space=pltpu.VMEM)

    K, V = pl.pallas_call(
        _kv_body,
        out_shape=(
            jax.ShapeDtypeStruct((S, D), _F32),
            jax.ShapeDtypeStruct((S, D), _F32),
        ),
        in_specs=[vmem] * 4,
        out_specs=(vmem, vmem),
        scratch_shapes=[
            pltpu.VMEM((3, DCS, PAY), jnp.bfloat16),
            pltpu.SemaphoreType.DMA((3,)),
            pltpu.SemaphoreType.DMA((3,)),
        ],
        compiler_params=pltpu.CompilerParams(
            collective_id=0, vmem_limit_bytes=34 * 1024 * 1024
        ),
    )(x, Wdkv, Wuk, Wuv)

    Q, Qr, Kr = pl.pallas_call(
        _proj_body,
        out_shape=(
            jax.ShapeDtypeStruct((S, D), _F32),
            jax.ShapeDtypeStruct((H, S, DR), _F32),
            jax.ShapeDtypeStruct((S, DR), _F32),
        ),
        in_specs=[vmem] * 4,
        out_specs=(vmem, vmem, vmem),
    )(x, Wq, Wqr, Wkr)

    O = pl.pallas_call(
        _attn_body,
        grid=(H,),
        out_shape=jax.ShapeDtypeStruct((S, D), _F32),
        in_specs=[
            pl.BlockSpec((S, DH), lambda h: (0, h)),
            pl.BlockSpec((S, DH), lambda h: (0, h)),
            pl.BlockSpec((S, DH), lambda h: (0, h)),
            pl.BlockSpec((1, S, DR), lambda h: (h, 0, 0)),
            pl.BlockSpec((S, DR), lambda h: (0, 0)),
        ],
        out_specs=pl.BlockSpec((S, DH), lambda h: (0, h)),
    )(Q, K, V, Qr, Kr)

    return pl.pallas_call(
        _out_body,
        out_shape=jax.ShapeDtypeStruct((1, S, D), _F32),
        in_specs=[vmem, vmem],
        out_specs=vmem,
    )(O, Wo)
